# baseline (speedup 1.0000x reference)
import functools

import jax
import jax.numpy as jnp
from jax import lax
from jax.experimental import pallas as pl
from jax.experimental.pallas import tpu as pltpu
from jax.experimental.pallas import tpu_sc as plsc

L = 512
IN = 300
MEM = 150
HID = 50
NCLS = 5
GW = 256


def _sc_gather(emb, idx):
    info = plsc.get_sparse_core_info()
    nc, ns = info.num_cores, info.num_subcores
    nw = nc * ns
    b = idx.shape[0]
    b_per_w = b // nw
    mesh = plsc.VectorSubcoreMesh(core_axis_name="c", subcore_axis_name="s")

    @functools.partial(
        pl.kernel,
        mesh=mesh,
        out_type=jax.ShapeDtypeStruct((b, IN), jnp.float32),
        scratch_types=[
            pltpu.VMEM((b_per_w,), jnp.int32),
            pltpu.VMEM((b_per_w, IN), jnp.float32),
            pltpu.SemaphoreType.DMA,
        ],
    )
    def gather_kernel(table_hbm, idx_hbm, out_hbm, idx_v, rows_v, sem):
        wid = lax.axis_index("s") * nc + lax.axis_index("c")
        base = wid * b_per_w
        pltpu.sync_copy(idx_hbm.at[pl.ds(base, b_per_w)], idx_v)
        copies = []
        for ch in range(b_per_w // 16):
            vals = idx_v[pl.ds(ch * 16, 16)]
            for k in range(16):
                i = ch * 16 + k
                row = vals[k]
                cp = pltpu.make_async_copy(
                    table_hbm.at[pl.ds(row, 1)], rows_v.at[pl.ds(i, 1)], sem)
                cp.start()
                copies.append(cp)
        for cp in copies:
            cp.wait()
        pltpu.sync_copy(rows_v, out_hbm.at[pl.ds(base, b_per_w)])

    return gather_kernel(emb, idx)


def _tc_body(x_ref, wx_ref, bx_ref, wh_ref, wa1_ref, wa2_ref, ba_ref,
             ww1_ref, ww2_ref, bwh_ref, wwp_ref, bwp_ref, out_ref,
             xw_ref, h_ref):
    xw_ref[...] = (jnp.dot(x_ref[...], wx_ref[...],
                           preferred_element_type=jnp.float32) + bx_ref[...])
    wh = wh_ref[...].astype(jnp.bfloat16)

    def substep(g2, c, h):
        g = g2 + jnp.dot(h.astype(jnp.bfloat16), wh,
                         preferred_element_type=jnp.float32)
        gi = jax.nn.sigmoid(g[:, 0:GW])
        go = jax.nn.sigmoid(g[:, GW:2 * GW])
        gu = jnp.tanh(g[:, 2 * GW:3 * GW])
        gf = jax.nn.sigmoid(g[:, 3 * GW:4 * GW])
        c = gi * gu + gf * c
        h = go * jnp.tanh(c)
        return c, h

    nb = L // 8

    def step(j, carry):
        cl, hl, cr, hr = carry
        jb = nb - 1 - j
        base = pl.multiple_of(8 * jb, 8)
        gl8 = xw_ref[pl.ds(base, 8), :]
        gr8 = xw_ref[pl.ds(L + base, 8), :]
        hls = [None] * 8
        hrs = [None] * 8
        for k in range(7, -1, -1):
            cl, hl = substep(gl8[k:k + 1], cl, hl)
            cr, hr = substep(gr8[k:k + 1], cr, hr)
            hls[k] = hl
            hrs[k] = hr
        h_ref[pl.ds(base, 8), :] = jnp.concatenate(hls, axis=0)
        h_ref[pl.ds(L + base, 8), :] = jnp.concatenate(hrs, axis=0)
        return cl, hl, cr, hr

    z = jnp.zeros((1, GW), jnp.float32)
    _, hl0, _, hr0 = lax.fori_loop(0, nb, step, (z, z, z, z))

    hall = h_ref[...]

    dn = (((1,), (1,)), ((), ()))
    s1 = lax.dot_general(hall, hl0, dn,
                         preferred_element_type=jnp.float32)
    s2 = lax.dot_general(hall, hr0, dn,
                         preferred_element_type=jnp.float32)
    rid = lax.broadcasted_iota(jnp.int32, (2 * L, 1), 0)
    is_r = rid >= L

    def msoftmax(s, mask):
        m = jnp.max(jnp.where(mask, s, -jnp.inf), axis=0, keepdims=True)
        e = jnp.where(mask, jnp.exp(s - m), 0.0)
        return e / jnp.sum(e, axis=0, keepdims=True)

    w_l = msoftmax(s1, is_r)
    w_r = msoftmax(s2, jnp.logical_not(is_r))
    beta0 = jnp.sum(w_l * hall, axis=0, keepdims=True)
    alpha0 = jnp.sum(w_r * hall, axis=0, keepdims=True)

    v_l = (jnp.dot(hl0, wa1_ref[...], preferred_element_type=jnp.float32)
           + jnp.dot(beta0, wa2_ref[...], preferred_element_type=jnp.float32)
           + ba_ref[...])
    v_r = (jnp.dot(hr0, wa1_ref[...], preferred_element_type=jnp.float32)
           + jnp.dot(alpha0, wa2_ref[...], preferred_element_type=jnp.float32)
           + ba_ref[...])

    mult = v_l * v_r
    absd = jnp.abs(v_l - v_r)
    hid = jax.nn.sigmoid(
        jnp.dot(mult, ww1_ref[...], preferred_element_type=jnp.float32)
        + jnp.dot(absd, ww2_ref[...], preferred_element_type=jnp.float32)
        + bwh_ref[...])
    logits = jnp.dot(hid, wwp_ref[...],
                     preferred_element_type=jnp.float32) + bwp_ref[...]
    m = jnp.max(logits, axis=1, keepdims=True)
    e = logits - m
    out_ref[...] = e - jnp.log(jnp.sum(jnp.exp(e), axis=1, keepdims=True))


def _pad_lanes(w, width=GW):
    return jnp.pad(w, ((0, 0), (0, width - w.shape[1])))


def kernel(linputs, rinputs, emb, W_ioux, b_ioux, W_iouh, b_iouh, W_fx, b_fx,
           W_fh, b_fh, W_attn, b_attn, W_wh, b_wh, W_wp, b_wp):
    idx = jnp.concatenate([linputs, rinputs]).astype(jnp.int32)
    x = _sc_gather(emb, idx)

    wx = jnp.concatenate([
        _pad_lanes(W_ioux[:, 0:MEM]),
        _pad_lanes(W_ioux[:, MEM:2 * MEM]),
        _pad_lanes(W_ioux[:, 2 * MEM:3 * MEM]),
        _pad_lanes(W_fx),
    ], axis=1)
    bxv = jnp.concatenate([
        jnp.pad(b_ioux[0:MEM] + b_iouh[0:MEM], (0, GW - MEM)),
        jnp.pad(b_ioux[MEM:2 * MEM] + b_iouh[MEM:2 * MEM], (0, GW - MEM)),
        jnp.pad(b_ioux[2 * MEM:3 * MEM] + b_iouh[2 * MEM:3 * MEM],
                (0, GW - MEM)),
        jnp.pad(b_fx + b_fh, (0, GW - MEM)),
    ])[None, :]
    wh = jnp.concatenate([
        _pad_lanes(W_iouh[:, 0:MEM]),
        _pad_lanes(W_iouh[:, MEM:2 * MEM]),
        _pad_lanes(W_iouh[:, 2 * MEM:3 * MEM]),
        _pad_lanes(W_fh),
    ], axis=1)
    wh = jnp.pad(wh, ((0, GW - MEM), (0, 0)))

    wa1 = jnp.pad(W_attn[:MEM], ((0, GW - MEM), (0, 0)))
    wa2 = jnp.pad(W_attn[MEM:], ((0, GW - MEM), (0, 0)))
    ba = b_attn[None, :]
    ww1 = W_wh[:MEM]
    ww2 = W_wh[MEM:]
    bwh = b_wh[None, :]
    bwp = b_wp[None, :]

    out = pl.pallas_call(
        _tc_body,
        out_shape=jax.ShapeDtypeStruct((1, NCLS), jnp.float32),
        scratch_shapes=[
            pltpu.VMEM((2 * L, 4 * GW), jnp.float32),
            pltpu.VMEM((2 * L, GW), jnp.float32),
        ],
    )(x, wx, bxv, wh, wa1, wa2, ba, ww1, ww2, bwh, W_wp, bwp)
    return out


# speedup vs baseline: 1.0345x; 1.0345x over previous
import functools

import jax
import jax.numpy as jnp
from jax import lax
from jax.experimental import pallas as pl
from jax.experimental.pallas import tpu as pltpu
from jax.experimental.pallas import tpu_sc as plsc

L = 512
IN = 300
MEM = 150
HID = 50
NCLS = 5
GW = 256


def _sc_gather(emb, idx):
    info = plsc.get_sparse_core_info()
    nc, ns = info.num_cores, info.num_subcores
    nw = nc * ns
    b = idx.shape[0]
    b_per_w = b // nw
    mesh = plsc.VectorSubcoreMesh(core_axis_name="c", subcore_axis_name="s")

    @functools.partial(
        pl.kernel,
        mesh=mesh,
        out_type=jax.ShapeDtypeStruct((b, IN), jnp.float32),
        scratch_types=[
            pltpu.VMEM((b_per_w,), jnp.int32),
            pltpu.VMEM((b_per_w, IN), jnp.float32),
            pltpu.SemaphoreType.DMA,
        ],
    )
    def gather_kernel(table_hbm, idx_hbm, out_hbm, idx_v, rows_v, sem):
        wid = lax.axis_index("s") * nc + lax.axis_index("c")
        base = wid * b_per_w
        pltpu.sync_copy(idx_hbm.at[pl.ds(base, b_per_w)], idx_v)
        copies = []
        for ch in range(b_per_w // 16):
            vals = idx_v[pl.ds(ch * 16, 16)]
            for k in range(16):
                i = ch * 16 + k
                row = vals[k]
                cp = pltpu.make_async_copy(
                    table_hbm.at[pl.ds(row, 1)], rows_v.at[pl.ds(i, 1)], sem)
                cp.start()
                copies.append(cp)
        for cp in copies:
            cp.wait()
        pltpu.sync_copy(rows_v, out_hbm.at[pl.ds(base, b_per_w)])

    return gather_kernel(emb, idx)


def _tc_body(x_ref, wx_ref, bx_ref, wh_ref, wa1_ref, wa2_ref, ba_ref,
             ww1_ref, ww2_ref, bwh_ref, wwp_ref, bwp_ref, out_ref,
             xw_ref, h_ref):
    xw_ref[...] = (jnp.dot(x_ref[...], wx_ref[...],
                           preferred_element_type=jnp.float32) + bx_ref[...])
    nb = L // 4

    def step(j, carry):
        c, h = carry
        jb = nb - 1 - j
        base = pl.multiple_of(8 * jb, 8)
        for k in range(3, -1, -1):
            g2 = jnp.concatenate([
                xw_ref[pl.ds(base + 2 * k, 1), :],
                xw_ref[pl.ds(base + 2 * k + 1, 1), :],
            ], axis=0)
            g = g2 + jnp.dot(h[:, 0:152].astype(jnp.bfloat16), wh_ref[...],
                             preferred_element_type=jnp.float32)
            gi = jax.nn.sigmoid(g[:, 0:GW])
            go = jax.nn.sigmoid(g[:, GW:2 * GW])
            gu = jnp.tanh(g[:, 2 * GW:3 * GW])
            gf = jax.nn.sigmoid(g[:, 3 * GW:4 * GW])
            c = gi * gu + gf * c
            h = go * jnp.tanh(c)
            h_ref[pl.ds(4 * jb + k, 1), :, :] = h[None]
        return c, h

    z = jnp.zeros((2, GW), jnp.float32)
    _, h_fin = lax.fori_loop(0, nb, step, (z, z))

    hl0 = h_fin[0:1, :]
    hr0 = h_fin[1:2, :]
    h3 = h_ref[...]

    s1 = jnp.sum(h3 * hl0[None], axis=2, keepdims=True)
    s2 = jnp.sum(h3 * hr0[None], axis=2, keepdims=True)
    bid = lax.broadcasted_iota(jnp.int32, (L, 2, 1), 1)
    is_r = bid == 1

    def msoftmax(s, mask):
        m = jnp.max(jnp.max(jnp.where(mask, s, -jnp.inf), axis=0,
                            keepdims=True), axis=1, keepdims=True)
        e = jnp.where(mask, jnp.exp(s - m), 0.0)
        d = jnp.sum(jnp.sum(e, axis=0, keepdims=True), axis=1,
                    keepdims=True)
        return e / d

    w_l = msoftmax(s1, is_r)
    w_r = msoftmax(s2, jnp.logical_not(is_r))
    beta0 = jnp.sum(jnp.sum(w_l * h3, axis=0), axis=0, keepdims=True)
    alpha0 = jnp.sum(jnp.sum(w_r * h3, axis=0), axis=0, keepdims=True)

    v_l = (jnp.dot(hl0, wa1_ref[...], preferred_element_type=jnp.float32)
           + jnp.dot(beta0, wa2_ref[...], preferred_element_type=jnp.float32)
           + ba_ref[...])
    v_r = (jnp.dot(hr0, wa1_ref[...], preferred_element_type=jnp.float32)
           + jnp.dot(alpha0, wa2_ref[...], preferred_element_type=jnp.float32)
           + ba_ref[...])

    mult = v_l * v_r
    absd = jnp.abs(v_l - v_r)
    hid = jax.nn.sigmoid(
        jnp.dot(mult, ww1_ref[...], preferred_element_type=jnp.float32)
        + jnp.dot(absd, ww2_ref[...], preferred_element_type=jnp.float32)
        + bwh_ref[...])
    logits = jnp.dot(hid, wwp_ref[...],
                     preferred_element_type=jnp.float32) + bwp_ref[...]
    m = jnp.max(logits, axis=1, keepdims=True)
    e = logits - m
    out_ref[...] = e - jnp.log(jnp.sum(jnp.exp(e), axis=1, keepdims=True))


def _pad_lanes(w, width=GW):
    return jnp.pad(w, ((0, 0), (0, width - w.shape[1])))


def kernel(linputs, rinputs, emb, W_ioux, b_ioux, W_iouh, b_iouh, W_fx, b_fx,
           W_fh, b_fh, W_attn, b_attn, W_wh, b_wh, W_wp, b_wp):
    idx = jnp.stack([linputs, rinputs], axis=1).reshape(-1).astype(jnp.int32)
    x = _sc_gather(emb, idx)

    wx = jnp.concatenate([
        _pad_lanes(W_ioux[:, 0:MEM]),
        _pad_lanes(W_ioux[:, MEM:2 * MEM]),
        _pad_lanes(W_ioux[:, 2 * MEM:3 * MEM]),
        _pad_lanes(W_fx),
    ], axis=1)
    bxv = jnp.concatenate([
        jnp.pad(b_ioux[0:MEM] + b_iouh[0:MEM], (0, GW - MEM)),
        jnp.pad(b_ioux[MEM:2 * MEM] + b_iouh[MEM:2 * MEM], (0, GW - MEM)),
        jnp.pad(b_ioux[2 * MEM:3 * MEM] + b_iouh[2 * MEM:3 * MEM],
                (0, GW - MEM)),
        jnp.pad(b_fx + b_fh, (0, GW - MEM)),
    ])[None, :]
    wh = jnp.concatenate([
        _pad_lanes(W_iouh[:, 0:MEM]),
        _pad_lanes(W_iouh[:, MEM:2 * MEM]),
        _pad_lanes(W_iouh[:, 2 * MEM:3 * MEM]),
        _pad_lanes(W_fh),
    ], axis=1)
    wh = jnp.pad(wh, ((0, 2), (0, 0))).astype(jnp.bfloat16)

    wa1 = jnp.pad(W_attn[:MEM], ((0, GW - MEM), (0, 0)))
    wa2 = jnp.pad(W_attn[MEM:], ((0, GW - MEM), (0, 0)))
    ba = b_attn[None, :]
    ww1 = W_wh[:MEM]
    ww2 = W_wh[MEM:]
    bwh = b_wh[None, :]
    bwp = b_wp[None, :]

    out = pl.pallas_call(
        _tc_body,
        out_shape=jax.ShapeDtypeStruct((1, NCLS), jnp.float32),
        scratch_shapes=[
            pltpu.VMEM((2 * L, 4 * GW), jnp.float32),
            pltpu.VMEM((L, 2, GW), jnp.float32),
        ],
    )(x, wx, bxv, wh, wa1, wa2, ba, ww1, ww2, bwh, W_wp, bwp)
    return out
